# speedup vs baseline: 1.0827x; 1.0827x over previous
import jax
import jax.numpy as jnp
from jax.experimental import pallas as pl
from jax.experimental.pallas import tpu as pltpu

H, W, P, D, EMB = 512, 512, 16, 128, 768
NR = H // P
NC = W // P
N_PATCH = NR * NC


def _add_kernel(x_ref, row_ref, col_ref, out_ref, pos_ref):
    @pl.when(pl.program_id(0) == 0)
    def _build():
        r = row_ref[...].reshape(NR, D // NR, EMB)[:, 2, :]
        c = col_ref[...].reshape(NC, D // NC, EMB)[:, 2, :]
        pos = r[:, None, :] + c[None, :, :]
        pos_ref[...] = pos.reshape(N_PATCH, EMB)

    out_ref[...] = x_ref[...] + pos_ref[...][None, :, :]


@jax.jit
def kernel(inputs, row_embedding, col_embedding):
    B = inputs.shape[0]
    bb = 2
    grid = (B // bb,)
    return pl.pallas_call(
        _add_kernel,
        grid=grid,
        in_specs=[
            pl.BlockSpec((bb, N_PATCH, EMB), lambda i: (i, 0, 0)),
            pl.BlockSpec((D, EMB), lambda i: (0, 0)),
            pl.BlockSpec((D, EMB), lambda i: (0, 0)),
        ],
        out_specs=pl.BlockSpec((bb, N_PATCH, EMB), lambda i: (i, 0, 0)),
        out_shape=jax.ShapeDtypeStruct(inputs.shape, inputs.dtype),
        scratch_shapes=[pltpu.VMEM((N_PATCH, EMB), jnp.float32)],
    )(inputs, row_embedding, col_embedding)


# speedup vs baseline: 1.0970x; 1.0132x over previous
import jax
import jax.numpy as jnp
from jax.experimental import pallas as pl
from jax.experimental.pallas import tpu as pltpu

H, W, P, D, EMB = 512, 512, 16, 128, 768
NR = H // P
NC = W // P
N_PATCH = NR * NC


def _add_kernel(x_ref, row_ref, col_ref, out_ref, pos_ref):
    @pl.when(pl.program_id(0) == 0)
    def _build():
        r = row_ref[...].reshape(NR, D // NR, EMB)[:, 2, :]
        c = col_ref[...].reshape(NC, D // NC, EMB)[:, 2, :]
        pos = r[:, None, :] + c[None, :, :]
        pos_ref[...] = pos.reshape(N_PATCH, EMB)

    out_ref[...] = x_ref[...] + pos_ref[...][None, :, :]


@jax.jit
def kernel(inputs, row_embedding, col_embedding):
    B = inputs.shape[0]
    bb = 4
    grid = (B // bb,)
    return pl.pallas_call(
        _add_kernel,
        grid=grid,
        in_specs=[
            pl.BlockSpec((bb, N_PATCH, EMB), lambda i: (i, 0, 0)),
            pl.BlockSpec((D, EMB), lambda i: (0, 0)),
            pl.BlockSpec((D, EMB), lambda i: (0, 0)),
        ],
        out_specs=pl.BlockSpec((bb, N_PATCH, EMB), lambda i: (i, 0, 0)),
        out_shape=jax.ShapeDtypeStruct(inputs.shape, inputs.dtype),
        scratch_shapes=[pltpu.VMEM((N_PATCH, EMB), jnp.float32)],
    )(inputs, row_embedding, col_embedding)
